# Initial kernel scaffold; baseline (speedup 1.0000x reference)
import jax
import jax.numpy as jnp
from jax.experimental import pallas as pl


def kernel(x, edge_index, W_l1, b_l1, W_r1, W_l2, b_l2, W_r2):
    raise NotImplementedError("write your pallas kernel here")


# speedup vs baseline: 3.0140x; 3.0140x over previous
import jax
import jax.numpy as jnp
from jax import lax
from jax.experimental import pallas as pl
from jax.experimental.pallas import tpu as pltpu
from jax.experimental.pallas import tpu_sc as plsc

N = 10000
E = 320000
D = 128
NC = 2
NS = 16
NW = NC * NS
NP = 10240
RPW = 80
NROWS = NW * RPW
EP = NROWS * 128
SUB_ROWS = NP // NS
ZROWS = 64

_SC_PARAMS = pltpu.CompilerParams(needs_layout_passes=False)
_MESH = plsc.VectorSubcoreMesh(core_axis_name="c", subcore_axis_name="s")


def _sc_segsum_body(x_hbm, srcI_hbm, dstI_hbm, acc_out, srcb, dstb, rows,
                    zbuf, acc_sh, sem):
    cid = lax.axis_index("c")
    sid = lax.axis_index("s")
    wid = sid * NC + cid

    zero16 = jnp.zeros((16,), jnp.float32)

    def zrow(r, _):
        for i in range(D // 16):
            zbuf[r, pl.ds(i * 16, 16)] = zero16
        return 0
    lax.fori_loop(0, ZROWS, zrow, 0)

    def zacc(t, _):
        pltpu.sync_copy(zbuf,
                        acc_sh.at[pl.ds(sid * SUB_ROWS + t * ZROWS, ZROWS)])
        return 0
    lax.fori_loop(0, SUB_ROWS // ZROWS, zacc, 0)

    base = wid * RPW
    pltpu.sync_copy(srcI_hbm.at[pl.ds(base, RPW)], srcb)
    pltpu.sync_copy(dstI_hbm.at[pl.ds(base, RPW)], dstb)
    plsc.subcore_barrier()

    def step(j, _):
        pltpu.async_copy(x_hbm.at[srcb.at[j]], rows, sem).wait()
        pltpu.sync_copy(rows, acc_sh.at[dstb.at[j]], add=True)
        return 0
    lax.fori_loop(0, RPW, step, 0)

    plsc.subcore_barrier()
    pltpu.sync_copy(acc_sh.at[pl.ds(sid * SUB_ROWS, SUB_ROWS)],
                    acc_out.at[cid].at[pl.ds(sid * SUB_ROWS, SUB_ROWS)])


_sc_segsum = pl.kernel(
    _sc_segsum_body,
    out_type=(jax.ShapeDtypeStruct((NC, NP, D), jnp.float32),),
    mesh=_MESH,
    scratch_types=[
        pltpu.VMEM((RPW, 128), jnp.int32),
        pltpu.VMEM((RPW, 128), jnp.int32),
        pltpu.VMEM((128, D), jnp.float32),
        pltpu.VMEM((ZROWS, D), jnp.float32),
        pltpu.VMEM_SHARED((NP, D), jnp.float32),
        pltpu.SemaphoreType.DMA,
    ],
    compiler_params=_SC_PARAMS,
)


def _sc_deg_body(dstI_hbm, deg_out, dstb, degv):
    cid = lax.axis_index("c")
    sid = lax.axis_index("s")
    wid = sid * NC + cid

    zero16 = jnp.zeros((16,), jnp.float32)
    ones16 = jnp.ones((16,), jnp.float32)

    def zdeg(k, _):
        degv[pl.ds(k * 16, 16)] = zero16
        return 0
    lax.fori_loop(0, NP // 16, zdeg, 0)

    pltpu.sync_copy(dstI_hbm.at[pl.ds(wid * RPW, RPW)], dstb)

    def step(j, _):
        def dstep(i, _):
            idx = dstb[j, pl.ds(i * 16, 16)]
            plsc.addupdate_scatter(degv, [idx], ones16)
            return 0
        lax.fori_loop(0, 128 // 16, dstep, 0)
        return 0
    lax.fori_loop(0, RPW, step, 0)

    pltpu.sync_copy(degv, deg_out.at[pl.ds(wid * NP, NP)])


_sc_deg = pl.kernel(
    _sc_deg_body,
    out_type=(jax.ShapeDtypeStruct((NW * NP,), jnp.float32),),
    mesh=_MESH,
    scratch_types=[
        pltpu.VMEM((RPW, 128), jnp.int32),
        pltpu.VMEM((NP,), jnp.float32),
    ],
    compiler_params=_SC_PARAMS,
)


def _tc_layer(acc, deg, h_in, W_l, b_l, W_r, relu):
    RB = 1024

    def tc_body(acc_ref, deg_ref, h_ref, wl_ref, b_ref, wr_ref, o_ref):
        s = acc_ref[0] + acc_ref[1]
        dsum = jnp.sum(deg_ref[...], axis=0)
        r = 1.0 / jnp.clip(dsum, 1.0, None)
        agg = s * r[:, None]
        y = (jnp.dot(agg, wl_ref[...], preferred_element_type=jnp.float32)
             + b_ref[...]
             + jnp.dot(h_ref[...], wr_ref[...],
                       preferred_element_type=jnp.float32))
        if relu:
            y = jnp.maximum(y, 0.0)
        o_ref[...] = y

    return pl.pallas_call(
        tc_body,
        grid=(NP // RB,),
        in_specs=[
            pl.BlockSpec((NC, RB, D), lambda i: (0, i, 0)),
            pl.BlockSpec((NW, RB), lambda i: (0, i)),
            pl.BlockSpec((RB, D), lambda i: (i, 0)),
            pl.BlockSpec((D, D), lambda i: (0, 0)),
            pl.BlockSpec((1, D), lambda i: (0, 0)),
            pl.BlockSpec((D, D), lambda i: (0, 0)),
        ],
        out_specs=pl.BlockSpec((RB, D), lambda i: (i, 0)),
        out_shape=jax.ShapeDtypeStruct((NP, D), jnp.float32),
    )(acc, deg, h_in, W_l, b_l.reshape(1, D), W_r)


def kernel(x, edge_index, W_l1, b_l1, W_r1, W_l2, b_l2, W_r2):
    src = edge_index[0]
    dst = edge_index[1]
    pad = EP - E
    srcp = jnp.concatenate(
        [src, jnp.zeros((pad,), jnp.int32)]).reshape(NROWS, 128)
    dstp = jnp.concatenate(
        [dst, jnp.full((pad,), N, jnp.int32)]).reshape(NROWS, 128)
    xp = jnp.pad(x, ((0, NP - N), (0, 0)))

    (deg,) = _sc_deg(dstp)
    deg = deg.reshape(NW, NP)
    (acc1,) = _sc_segsum(xp, srcp, dstp)
    h = _tc_layer(acc1, deg, xp, W_l1, b_l1, W_r1, relu=True)
    (acc2,) = _sc_segsum(h, srcp, dstp)
    out = _tc_layer(acc2, deg, h, W_l2, b_l2, W_r2, relu=False)
    return out[:N]


# speedup vs baseline: 3.2661x; 1.0836x over previous
import jax
import jax.numpy as jnp
from jax import lax
from jax.experimental import pallas as pl
from jax.experimental.pallas import tpu as pltpu
from jax.experimental.pallas import tpu_sc as plsc

N = 10000
E = 320000
D = 128
NC = 2
NS = 16
NW = NC * NS
NP = 10240
RPW = 80
NROWS = NW * RPW
EP = NROWS * 128
SUB_ROWS = NP // NS
ZROWS = 64

_SC_PARAMS = pltpu.CompilerParams(needs_layout_passes=False)
_MESH = plsc.VectorSubcoreMesh(core_axis_name="c", subcore_axis_name="s")


IC = 16
NCH = RPW // IC
SLOTS = 3


def _sc_segsum_body(x_hbm, srcI_hbm, dstI_hbm, acc_out, srcb, dstb,
                    rows0, rows1, acc_sh, gsem0, gsem1, ssem0, ssem1, isem):
    rows = [rows0, rows1]
    gsem = [gsem0, gsem1]
    ssem = [ssem0, ssem1]
    cid = lax.axis_index("c")
    sid = lax.axis_index("s")
    wid = sid * NC + cid
    base = wid * RPW

    zero16 = jnp.zeros((16,), jnp.float32)

    def zrow(r, _):
        for i in range(D // 16):
            rows0[r, pl.ds(i * 16, 16)] = zero16
        return 0
    lax.fori_loop(0, 128, zrow, 0)

    def zacc(t, _):
        pltpu.sync_copy(rows0,
                        acc_sh.at[pl.ds(sid * SUB_ROWS + t * 128, 128)])
        return 0
    lax.fori_loop(0, SUB_ROWS // 128, zacc, 0)

    pltpu.sync_copy(srcI_hbm.at[pl.ds(base, IC)], srcb.at[pl.ds(0, IC)])
    pltpu.sync_copy(dstI_hbm.at[pl.ds(base, IC)], dstb.at[pl.ds(0, IC)])
    plsc.subcore_barrier()

    def ipos(j):
        return ((j // IC) % SLOTS) * IC + (j % IC)

    def gfire(j, b):
        pltpu.async_copy(x_hbm.at[srcb.at[ipos(j)]], rows[b], gsem[b])

    def gwait(j, b):
        pltpu.make_async_copy(x_hbm.at[srcb.at[ipos(j)]], rows[b],
                              gsem[b]).wait()

    def sfire(j, b):
        pltpu.async_copy(rows[b], acc_sh.at[dstb.at[ipos(j)]], ssem[b],
                         add=True)

    def swait(j, b):
        pltpu.make_async_copy(rows[b], acc_sh.at[dstb.at[ipos(j)]],
                              ssem[b]).wait()

    gfire(0, 0)

    def chunk(c, _):
        c16 = c * IC
        slot = ((c + 1) % SLOTS) * IC

        @pl.when(c + 1 < NCH)
        def _():
            pltpu.async_copy(srcI_hbm.at[pl.ds(base + c16 + IC, IC)],
                             srcb.at[pl.ds(slot, IC)], isem)
            pltpu.async_copy(dstI_hbm.at[pl.ds(base + c16 + IC, IC)],
                             dstb.at[pl.ds(slot, IC)], isem)

        def pair(j2, _):
            for k in range(2):
                j = c16 + j2 * 2 + k
                gwait(j, k)

                @pl.when(j >= 1)
                def _():
                    swait(j, 1 - k)
                if k == 0:
                    gfire(j + 1, 1)
                else:
                    @pl.when(j2 < IC // 2 - 1)
                    def _():
                        gfire(j + 1, 0)
                sfire(j, k)
            return 0
        lax.fori_loop(0, IC // 2, pair, 0)

        @pl.when(c + 1 < NCH)
        def _():
            pltpu.make_async_copy(srcI_hbm.at[pl.ds(base + c16 + IC, IC)],
                                  srcb.at[pl.ds(slot, IC)], isem).wait()
            pltpu.make_async_copy(dstI_hbm.at[pl.ds(base + c16 + IC, IC)],
                                  dstb.at[pl.ds(slot, IC)], isem).wait()
            gfire(c16 + IC, 0)
        return 0
    lax.fori_loop(0, NCH, chunk, 0)

    swait(RPW - 1, 1)
    plsc.subcore_barrier()
    pltpu.sync_copy(acc_sh.at[pl.ds(sid * SUB_ROWS, SUB_ROWS)],
                    acc_out.at[cid].at[pl.ds(sid * SUB_ROWS, SUB_ROWS)])


_sc_segsum = pl.kernel(
    _sc_segsum_body,
    out_type=(jax.ShapeDtypeStruct((NC, NP, D), jnp.float32),),
    mesh=_MESH,
    scratch_types=[
        pltpu.VMEM((SLOTS * IC, 128), jnp.int32),
        pltpu.VMEM((SLOTS * IC, 128), jnp.int32),
        pltpu.VMEM((128, D), jnp.float32),
        pltpu.VMEM((128, D), jnp.float32),
        pltpu.VMEM_SHARED((NP, D), jnp.float32),
        pltpu.SemaphoreType.DMA,
        pltpu.SemaphoreType.DMA,
        pltpu.SemaphoreType.DMA,
        pltpu.SemaphoreType.DMA,
        pltpu.SemaphoreType.DMA,
    ],
    compiler_params=_SC_PARAMS,
)


def _sc_deg_body(dstI_hbm, deg_out, dstb, degv):
    cid = lax.axis_index("c")
    sid = lax.axis_index("s")
    wid = sid * NC + cid

    zero16 = jnp.zeros((16,), jnp.float32)
    ones16 = jnp.ones((16,), jnp.float32)

    def zdeg(k, _):
        degv[pl.ds(k * 16, 16)] = zero16
        return 0
    lax.fori_loop(0, NP // 16, zdeg, 0)

    pltpu.sync_copy(dstI_hbm.at[pl.ds(wid * RPW, RPW)], dstb)

    def step(j, _):
        def dstep(i, _):
            idx = dstb[j, pl.ds(i * 16, 16)]
            plsc.addupdate_scatter(degv, [idx], ones16)
            return 0
        lax.fori_loop(0, 128 // 16, dstep, 0)
        return 0
    lax.fori_loop(0, RPW, step, 0)

    pltpu.sync_copy(degv, deg_out.at[pl.ds(wid * NP, NP)])


_sc_deg = pl.kernel(
    _sc_deg_body,
    out_type=(jax.ShapeDtypeStruct((NW * NP,), jnp.float32),),
    mesh=_MESH,
    scratch_types=[
        pltpu.VMEM((RPW, 128), jnp.int32),
        pltpu.VMEM((NP,), jnp.float32),
    ],
    compiler_params=_SC_PARAMS,
)


def _tc_layer(acc, deg, h_in, W_l, b_l, W_r, relu):
    RB = 1024

    def tc_body(acc_ref, deg_ref, h_ref, wl_ref, b_ref, wr_ref, o_ref):
        s = acc_ref[0] + acc_ref[1]
        dsum = jnp.sum(deg_ref[...], axis=0)
        r = 1.0 / jnp.clip(dsum, 1.0, None)
        agg = s * r[:, None]
        y = (jnp.dot(agg, wl_ref[...], preferred_element_type=jnp.float32)
             + b_ref[...]
             + jnp.dot(h_ref[...], wr_ref[...],
                       preferred_element_type=jnp.float32))
        if relu:
            y = jnp.maximum(y, 0.0)
        o_ref[...] = y

    return pl.pallas_call(
        tc_body,
        grid=(NP // RB,),
        in_specs=[
            pl.BlockSpec((NC, RB, D), lambda i: (0, i, 0)),
            pl.BlockSpec((NW, RB), lambda i: (0, i)),
            pl.BlockSpec((RB, D), lambda i: (i, 0)),
            pl.BlockSpec((D, D), lambda i: (0, 0)),
            pl.BlockSpec((1, D), lambda i: (0, 0)),
            pl.BlockSpec((D, D), lambda i: (0, 0)),
        ],
        out_specs=pl.BlockSpec((RB, D), lambda i: (i, 0)),
        out_shape=jax.ShapeDtypeStruct((NP, D), jnp.float32),
    )(acc, deg, h_in, W_l, b_l.reshape(1, D), W_r)


def kernel(x, edge_index, W_l1, b_l1, W_r1, W_l2, b_l2, W_r2):
    src = edge_index[0]
    dst = edge_index[1]
    pad = EP - E
    srcp = jnp.concatenate(
        [src, jnp.zeros((pad,), jnp.int32)]).reshape(NROWS, 128)
    pad_dst = N + jnp.arange(pad, dtype=jnp.int32) % (NP - N)
    dstp = jnp.concatenate([dst, pad_dst]).reshape(NROWS, 128)
    xp = jnp.pad(x, ((0, NP - N), (0, 0)))

    (deg,) = _sc_deg(dstp)
    deg = deg.reshape(NW, NP)
    (acc1,) = _sc_segsum(xp, srcp, dstp)
    h = _tc_layer(acc1, deg, xp, W_l1, b_l1, W_r1, relu=True)
    (acc2,) = _sc_segsum(h, srcp, dstp)
    out = _tc_layer(acc2, deg, h, W_l2, b_l2, W_r2, relu=False)
    return out[:N]
